# baseline (speedup 1.0000x reference)
import functools

import jax
import jax.numpy as jnp
from jax import lax
from jax.experimental import pallas as pl
from jax.experimental.pallas import tpu as pltpu
from jax.experimental.pallas import tpu_sc as plsc

DIM = 64
LANES = 16
BBLK = 128
SGRP = 4


@functools.lru_cache(maxsize=None)
def _build(batch: int, seq: int):
    info = plsc.get_sparse_core_info()
    num_workers = info.num_cores * info.num_subcores
    assert batch == num_workers * BBLK and seq % (2 * SGRP) == 0
    n_tiles = DIM // 8
    half = seq // (2 * SGRP)

    mesh = plsc.VectorSubcoreMesh(core_axis_name="c", subcore_axis_name="s")

    @functools.partial(
        pl.kernel,
        mesh=mesh,
        compiler_params=pltpu.CompilerParams(use_tc_tiling_on_sc=False,
                                             needs_layout_passes=False),
        out_type=jax.ShapeDtypeStruct(
            (seq, n_tiles, batch // BBLK, 8, BBLK), jnp.float32),
        scratch_types=[
            [pltpu.VMEM((SGRP, BBLK), jnp.int32) for _ in range(2)],
            [pltpu.VMEM((SGRP, BBLK), jnp.int32) for _ in range(2)],
            [pltpu.VMEM((SGRP, BBLK), jnp.int32) for _ in range(2)],
            [pltpu.VMEM((BBLK, DIM), jnp.float32) for _ in range(2)],
            [pltpu.VMEM((BBLK, DIM), jnp.float32) for _ in range(2)],
            [pltpu.VMEM((BBLK, DIM), jnp.float32) for _ in range(2)],
            [pltpu.VMEM((n_tiles, SGRP, 8, BBLK), jnp.float32)
             for _ in range(2)],
            [pltpu.SemaphoreType.DMA for _ in range(2)],
            [pltpu.SemaphoreType.DMA for _ in range(2)],
            [pltpu.SemaphoreType.DMA for _ in range(2)],
        ],
    )
    def emb_sum(gid, cid, rid, basic, homo, rna, out,
                gi_v, ci_v, ri_v, a_v, h_v, r_v, st_v,
                sem_idx, sem_g, sem_st):
        w = lax.axis_index("s") * info.num_cores + lax.axis_index("c")
        wcol = w * BBLK

        def fire_idx(g, b):
            src = (pl.ds(g * SGRP, SGRP), pl.ds(wcol, BBLK))
            pltpu.async_copy(gid.at[src], gi_v[b], sem_idx[b])
            pltpu.async_copy(cid.at[src], ci_v[b], sem_idx[b])
            pltpu.async_copy(rid.at[src], ri_v[b], sem_idx[b])

        def wait_idx(b):
            for _ in range(3):
                pltpu.make_async_copy(
                    gid.at[pl.ds(0, SGRP), pl.ds(0, BBLK)], gi_v[b],
                    sem_idx[b]).wait()

        def fire_gathers(q, b, j):
            pltpu.async_copy(basic.at[gi_v[b].at[j]], a_v[q], sem_g[q])
            pltpu.async_copy(homo.at[ci_v[b].at[j]], h_v[q], sem_g[q])
            pltpu.async_copy(rna.at[ri_v[b].at[j]], r_v[q], sem_g[q])

        def wait_gathers(q):
            for _ in range(3):
                pltpu.make_async_copy(basic.at[gi_v[0].at[0]], a_v[q],
                                      sem_g[q]).wait()

        def fire_stores(g, pg):
            for dt in range(n_tiles):
                pltpu.async_copy(st_v[pg].at[dt],
                                 out.at[pl.ds(g * SGRP, SGRP), dt, w],
                                 sem_st[pg])

        def wait_stores(pg):
            for _ in range(n_tiles):
                pltpu.make_async_copy(st_v[pg].at[0],
                                      out.at[pl.ds(0, SGRP), 0, 0],
                                      sem_st[pg]).wait()

        def compute(q, pg, sl):
            a, h, r, st = a_v[q], h_v[q], r_v[q], st_v[pg]
            iota = lax.iota(jnp.int32, LANES)

            @plsc.parallel_loop(0, (BBLK // LANES) * DIM, step=1, unroll=4)
            def vec(v):
                col = v >> 3
                bg = v & 7
                dt = v >> 6
                di = (v >> 3) & 7
                rows16 = bg * LANES + iota
                colv = jnp.full((LANES,), col, jnp.int32)
                val = (plsc.load_gather(a, [rows16, colv])
                       + plsc.load_gather(h, [rows16, colv])
                       + plsc.load_gather(r, [rows16, colv]))
                st[dt, sl, di, pl.ds(bg * LANES, LANES)] = val

        fire_idx(0, 0)
        fire_idx(1, 1)
        wait_idx(0)
        fire_gathers(0, 0, 0)

        def body(t, carry):
            for k in range(2 * SGRP):
                q = k % 2
                pg = k // SGRP
                sl = k % SGRP
                wait_gathers(q)
                if k == SGRP - 1:
                    wait_idx(1)

                    @pl.when(t < half - 1)
                    def _():
                        fire_idx(2 * t + 2, 0)
                if k == 2 * SGRP - 1:
                    @pl.when(t < half - 1)
                    def _():
                        wait_idx(0)
                        fire_gathers(1 - q, 0, 0)
                        fire_idx(2 * t + 3, 1)
                else:
                    nk = k + 1
                    fire_gathers(1 - q, nk // SGRP, nk % SGRP)
                if k == 0 or k == SGRP:
                    @pl.when(t > 0)
                    def _():
                        wait_stores(pg)
                compute(q, pg, sl)
                if k == SGRP - 1:
                    fire_stores(2 * t, 0)
                if k == 2 * SGRP - 1:
                    fire_stores(2 * t + 1, 1)
            return carry

        lax.fori_loop(0, half, body, 0)
        wait_stores(0)
        wait_stores(1)

    return emb_sum


def kernel(x_gene_id, x_connect_id, x_rna_type, basic_table, homo_table, rna_table):
    batch, seq = x_gene_id.shape
    gid = x_gene_id.T.astype(jnp.int32)
    cid = x_connect_id.T.astype(jnp.int32)
    rid = x_rna_type.T.astype(jnp.int32)
    out5 = _build(batch, seq)(gid, cid, rid, basic_table, homo_table,
                              rna_table)
    return out5.transpose(2, 4, 0, 1, 3).reshape(batch, seq, DIM)


# speedup vs baseline: 1.9337x; 1.9337x over previous
import functools

import jax
import jax.numpy as jnp
from jax import lax
from jax.experimental import pallas as pl
from jax.experimental.pallas import tpu as pltpu
from jax.experimental.pallas import tpu_sc as plsc

DIM = 64
LANES = 16
BBLK = 128
SGRP = 4


@functools.lru_cache(maxsize=None)
def _build(batch: int, seq: int):
    info = plsc.get_sparse_core_info()
    num_workers = info.num_cores * info.num_subcores
    assert batch == num_workers * BBLK and seq % (2 * SGRP) == 0
    n_tiles = DIM // 8
    half = seq // (2 * SGRP)

    mesh = plsc.VectorSubcoreMesh(core_axis_name="c", subcore_axis_name="s")

    @functools.partial(
        pl.kernel,
        mesh=mesh,
        compiler_params=pltpu.CompilerParams(use_tc_tiling_on_sc=False,
                                             needs_layout_passes=False),
        out_type=jax.ShapeDtypeStruct(
            (seq, n_tiles, batch // BBLK, 8, BBLK), jnp.float32),
        scratch_types=[
            [pltpu.VMEM((SGRP, BBLK), jnp.int32) for _ in range(2)],
            [pltpu.VMEM((SGRP, BBLK), jnp.int32) for _ in range(2)],
            [pltpu.VMEM((SGRP, BBLK), jnp.int32) for _ in range(2)],
            [pltpu.VMEM((BBLK, DIM), jnp.float32) for _ in range(2)],
            [pltpu.VMEM((BBLK, DIM), jnp.float32) for _ in range(2)],
            [pltpu.VMEM((BBLK, DIM), jnp.float32) for _ in range(2)],
            [pltpu.VMEM((n_tiles, SGRP, 8, BBLK + 1), jnp.float32)
             for _ in range(2)],
            [pltpu.SemaphoreType.DMA for _ in range(2)],
            [pltpu.SemaphoreType.DMA for _ in range(2)],
            [pltpu.SemaphoreType.DMA for _ in range(2)],
        ],
    )
    def emb_sum(gid, cid, rid, basic, homo, rna, out,
                gi_v, ci_v, ri_v, a_v, h_v, r_v, st_v,
                sem_idx, sem_g, sem_st):
        w = lax.axis_index("s") * info.num_cores + lax.axis_index("c")
        wcol = w * BBLK

        def fire_idx(g, b):
            src = (pl.ds(g * SGRP, SGRP), pl.ds(wcol, BBLK))
            pltpu.async_copy(gid.at[src], gi_v[b], sem_idx[b])
            pltpu.async_copy(cid.at[src], ci_v[b], sem_idx[b])
            pltpu.async_copy(rid.at[src], ri_v[b], sem_idx[b])

        def wait_idx(b):
            for _ in range(3):
                pltpu.make_async_copy(
                    gid.at[pl.ds(0, SGRP), pl.ds(0, BBLK)], gi_v[b],
                    sem_idx[b]).wait()

        def fire_gathers(q, b, j):
            pltpu.async_copy(basic.at[gi_v[b].at[j]], a_v[q], sem_g[q])
            pltpu.async_copy(homo.at[ci_v[b].at[j]], h_v[q], sem_g[q])
            pltpu.async_copy(rna.at[ri_v[b].at[j]], r_v[q], sem_g[q])

        def wait_gathers(q):
            for _ in range(3):
                pltpu.make_async_copy(basic.at[gi_v[0].at[0]], a_v[q],
                                      sem_g[q]).wait()

        def fire_stores(g, pg):
            for dt in range(n_tiles):
                pltpu.async_copy(st_v[pg].at[dt, :, :, pl.ds(0, BBLK)],
                                 out.at[pl.ds(g * SGRP, SGRP), dt, w],
                                 sem_st[pg])

        def wait_stores(pg):
            for _ in range(n_tiles):
                pltpu.make_async_copy(st_v[pg].at[0, :, :, pl.ds(0, BBLK)],
                                      out.at[pl.ds(0, SGRP), 0, 0],
                                      sem_st[pg]).wait()

        def compute(q, pg, sl):
            a, h, r, st = a_v[q], h_v[q], r_v[q], st_v[pg]
            iota = lax.iota(jnp.int32, LANES)
            slv = jnp.full((LANES,), sl, jnp.int32)
            dts = []
            dis = []
            for c in range(DIM // LANES):
                dvec = c * LANES + iota
                dts.append(dvec >> 3)
                dis.append(dvec & 7)

            @plsc.parallel_loop(0, BBLK, step=1, unroll=2)
            def rowf(rr):
                rrv = jnp.full((LANES,), rr, jnp.int32)
                for c in range(DIM // LANES):
                    sl16 = pl.ds(c * LANES, LANES)
                    val = a[rr, sl16] + h[rr, sl16] + r[rr, sl16]
                    plsc.store_scatter(st, [dts[c], slv, dis[c], rrv], val)

        fire_idx(0, 0)
        fire_idx(1, 1)
        wait_idx(0)
        fire_gathers(0, 0, 0)

        def body(t, carry):
            for k in range(2 * SGRP):
                q = k % 2
                pg = k // SGRP
                sl = k % SGRP
                wait_gathers(q)
                if k == SGRP - 1:
                    wait_idx(1)

                    @pl.when(t < half - 1)
                    def _():
                        fire_idx(2 * t + 2, 0)
                if k == 2 * SGRP - 1:
                    @pl.when(t < half - 1)
                    def _():
                        wait_idx(0)
                        fire_gathers(1 - q, 0, 0)
                        fire_idx(2 * t + 3, 1)
                else:
                    nk = k + 1
                    fire_gathers(1 - q, nk // SGRP, nk % SGRP)
                if k == 0 or k == SGRP:
                    @pl.when(t > 0)
                    def _():
                        wait_stores(pg)
                compute(q, pg, sl)
                if k == SGRP - 1:
                    fire_stores(2 * t, 0)
                if k == 2 * SGRP - 1:
                    fire_stores(2 * t + 1, 1)
            return carry

        lax.fori_loop(0, half, body, 0)
        wait_stores(0)
        wait_stores(1)

    return emb_sum


def kernel(x_gene_id, x_connect_id, x_rna_type, basic_table, homo_table, rna_table):
    batch, seq = x_gene_id.shape
    gid = x_gene_id.T.astype(jnp.int32)
    cid = x_connect_id.T.astype(jnp.int32)
    rid = x_rna_type.T.astype(jnp.int32)
    out5 = _build(batch, seq)(gid, cid, rid, basic_table, homo_table,
                              rna_table)
    return out5.transpose(2, 4, 0, 1, 3).reshape(batch, seq, DIM)


# speedup vs baseline: 3.1455x; 1.6267x over previous
import functools

import jax
import jax.numpy as jnp
from jax import lax
from jax.experimental import pallas as pl
from jax.experimental.pallas import tpu as pltpu
from jax.experimental.pallas import tpu_sc as plsc

DIM = 64
LANES = 16
BBLK = 128
SGRP = 4


@functools.lru_cache(maxsize=None)
def _build(batch: int, seq: int):
    info = plsc.get_sparse_core_info()
    num_workers = info.num_cores * info.num_subcores
    assert batch == num_workers * BBLK and seq % (2 * SGRP) == 0
    n_tiles = DIM // 8
    half = seq // (2 * SGRP)

    mesh = plsc.VectorSubcoreMesh(core_axis_name="c", subcore_axis_name="s")

    @functools.partial(
        pl.kernel,
        mesh=mesh,
        compiler_params=pltpu.CompilerParams(use_tc_tiling_on_sc=False,
                                             needs_layout_passes=False),
        out_type=jax.ShapeDtypeStruct(
            (seq, n_tiles, batch // BBLK, 8, BBLK), jnp.float32),
        scratch_types=[
            [pltpu.VMEM((SGRP, BBLK), jnp.int32) for _ in range(2)],
            [pltpu.VMEM((SGRP, BBLK), jnp.int32) for _ in range(2)],
            [pltpu.VMEM((SGRP, BBLK), jnp.int32) for _ in range(2)],
            [pltpu.VMEM((BBLK, DIM), jnp.float32) for _ in range(2)],
            [pltpu.VMEM((BBLK, DIM), jnp.float32) for _ in range(2)],
            pltpu.VMEM((51, DIM + 1), jnp.float32),
            [pltpu.VMEM((n_tiles, SGRP, 8, BBLK + 1), jnp.float32)
             for _ in range(2)],
            [pltpu.SemaphoreType.DMA for _ in range(2)],
            [pltpu.SemaphoreType.DMA for _ in range(2)],
            [pltpu.SemaphoreType.DMA for _ in range(2)],
        ],
    )
    def emb_sum(gid, cid, rid, basic, homo, rna, out,
                gi_v, ci_v, ri_v, a_v, h_v, rna_vm, st_v,
                sem_idx, sem_g, sem_st):
        w = lax.axis_index("s") * info.num_cores + lax.axis_index("c")
        wcol = w * BBLK

        def fire_idx(g, b):
            src = (pl.ds(g * SGRP, SGRP), pl.ds(wcol, BBLK))
            pltpu.async_copy(gid.at[src], gi_v[b], sem_idx[b])
            pltpu.async_copy(cid.at[src], ci_v[b], sem_idx[b])
            pltpu.async_copy(rid.at[src], ri_v[b], sem_idx[b])

        def wait_idx(b):
            for _ in range(3):
                pltpu.make_async_copy(
                    gid.at[pl.ds(0, SGRP), pl.ds(0, BBLK)], gi_v[b],
                    sem_idx[b]).wait()

        def fire_gathers(q, b, j):
            pltpu.async_copy(basic.at[gi_v[b].at[j]], a_v[q], sem_g[q])
            pltpu.async_copy(homo.at[ci_v[b].at[j]], h_v[q], sem_g[q])

        def wait_gathers(q):
            for _ in range(2):
                pltpu.make_async_copy(basic.at[gi_v[0].at[0]], a_v[q],
                                      sem_g[q]).wait()

        def fire_stores(g, pg):
            for dt in range(n_tiles):
                pltpu.async_copy(st_v[pg].at[dt, :, :, pl.ds(0, BBLK)],
                                 out.at[pl.ds(g * SGRP, SGRP), dt, w],
                                 sem_st[pg])

        def wait_stores(pg):
            for _ in range(n_tiles):
                pltpu.make_async_copy(st_v[pg].at[0, :, :, pl.ds(0, BBLK)],
                                      out.at[pl.ds(0, SGRP), 0, 0],
                                      sem_st[pg]).wait()

        def compute(q, pg, sl):
            a, h, st, ri = a_v[q], h_v[q], st_v[pg], ri_v[pg]
            iota = lax.iota(jnp.int32, LANES)
            slv = jnp.full((LANES,), sl, jnp.int32)
            dts = []
            dis = []
            for c in range(DIM // LANES):
                dvec = c * LANES + iota
                dts.append(dvec >> 3)
                dis.append(dvec & 7)

            @plsc.parallel_loop(0, BBLK, step=1, unroll=2)
            def rowf(rr):
                rrv = jnp.full((LANES,), rr, jnp.int32)
                for c in range(DIM // LANES):
                    sl16 = pl.ds(c * LANES, LANES)
                    val = a[rr, sl16] + h[rr, sl16]
                    plsc.store_scatter(st, [dts[c], slv, dis[c], rrv], val)

            def bgf(bg, carry):
                rid16 = ri[sl, pl.ds(bg * LANES, LANES)]
                bsl = pl.ds(bg * LANES, LANES)

                @plsc.parallel_loop(0, DIM, step=1, unroll=4)
                def df(d):
                    rn = plsc.load_gather(
                        rna_vm, [rid16, jnp.full((LANES,), d, jnp.int32)])
                    plsc.addupdate(st.at[d >> 3, sl, d & 7, bsl], rn)
                return carry

            lax.fori_loop(0, BBLK // LANES, bgf, 0)

        pltpu.sync_copy(rna, rna_vm.at[:, pl.ds(0, DIM)])

        fire_idx(0, 0)
        fire_idx(1, 1)
        wait_idx(0)
        fire_gathers(0, 0, 0)

        def body(t, carry):
            for k in range(2 * SGRP):
                q = k % 2
                pg = k // SGRP
                sl = k % SGRP
                wait_gathers(q)
                if k == SGRP - 1:
                    wait_idx(1)
                if k == SGRP:
                    @pl.when(t < half - 1)
                    def _():
                        fire_idx(2 * t + 2, 0)
                if k == 2 * SGRP - 1:
                    @pl.when(t < half - 1)
                    def _():
                        wait_idx(0)
                        fire_gathers(1 - q, 0, 0)
                else:
                    nk = k + 1
                    fire_gathers(1 - q, nk // SGRP, nk % SGRP)
                if k == 0 or k == SGRP:
                    @pl.when(t > 0)
                    def _():
                        wait_stores(pg)
                compute(q, pg, sl)
                if k == SGRP - 1:
                    fire_stores(2 * t, 0)
                if k == 2 * SGRP - 1:
                    fire_stores(2 * t + 1, 1)

                    @pl.when(t < half - 1)
                    def _():
                        fire_idx(2 * t + 3, 1)
            return carry

        lax.fori_loop(0, half, body, 0)
        wait_stores(0)
        wait_stores(1)

    return emb_sum


def kernel(x_gene_id, x_connect_id, x_rna_type, basic_table, homo_table, rna_table):
    batch, seq = x_gene_id.shape
    gid = x_gene_id.T.astype(jnp.int32)
    cid = x_connect_id.T.astype(jnp.int32)
    rid = x_rna_type.T.astype(jnp.int32)
    out5 = _build(batch, seq)(gid, cid, rid, basic_table, homo_table,
                              rna_table)
    return out5.transpose(2, 4, 0, 1, 3).reshape(batch, seq, DIM)
